# baseline (speedup 1.0000x reference)
import math

import numpy as np
import jax
import jax.numpy as jnp
from jax import lax
from jax.experimental import pallas as pl
from jax.experimental.pallas import tpu as pltpu

B = 4
L = 2048
S = 2048
D_IN = 1024
HIDDEN = 1024
H = 16
E = HIDDEN // H
FACTOR = 5
NJ = HIDDEN // E
RPH = L // NJ
U = min(FACTOR * int(np.ceil(np.log(L))), L)
GH = 4
ROWS = GH * RPH


def _body(kidx_ref, q_ref, k_ref, v_ref, wq_ref, wk_ref, wv_ref, wo_ref,
          out_ref, ks, vs, ctx, sel):
    f32 = jnp.float32
    bf16 = jnp.bfloat16
    qp = jnp.dot(q_ref[0], wq_ref[...], preferred_element_type=f32)
    kp = jnp.dot(k_ref[0], wk_ref[...], preferred_element_type=f32)
    vp = jnp.dot(v_ref[0].astype(bf16), wv_ref[...].astype(bf16),
                 preferred_element_type=f32)

    for g in range(GH):
        rs = slice(RPH * g, RPH * (g + 1))
        for j in range(NJ):
            sl = slice(L * g + RPH * j, L * g + RPH * (j + 1))
            cs = slice(E * j, E * (j + 1))
            ks[sl, :] = kp[rs, cs]

    mrows = []
    for g in range(GH):
        rs = slice(RPH * g, RPH * (g + 1))
        ksamp = jnp.concatenate(
            [ks[pl.ds(L * g + kidx_ref[t], 1), :] for t in range(U)], axis=0)
        qks = jnp.concatenate(
            [lax.dot_general(ksamp, qp[rs, E * j: E * (j + 1)],
                             (((1,), (1,)), ((), ())),
                             preferred_element_type=f32)
             for j in range(NJ)], axis=1)
        mrows.append(jnp.max(qks, axis=0, keepdims=True)
                     - jnp.sum(qks, axis=0, keepdims=True) * f32(1.0 / S))
    cur = jnp.concatenate(mrows, axis=0)

    rio = lax.broadcasted_iota(jnp.int32, (RPH, RPH), 0)
    cio = lax.broadcasted_iota(jnp.int32, (RPH, RPH), 1)
    stril = jnp.where(rio > cio, f32(1.0), f32(0.0))
    neg = f32(-jnp.inf)
    tacc = [jnp.zeros((RPH, E), f32) for _ in range(GH)]
    run2 = [jnp.zeros((RPH, E), f32) for _ in range(GH)]
    pref = [None] * GH
    for t in range(U):
        mval = jnp.max(cur, axis=1, keepdims=True)
        oh = cur == mval
        for g in range(GH):
            sel[U * g + t: U * g + t + 1, :] = jnp.where(
                oh[g:g + 1, :], f32(1.0), f32(0.0))
        cur = jnp.where(oh, neg, cur)
        for g in range(GH):
            rs = slice(RPH * g, RPH * (g + 1))
            if t < NJ:
                cs = slice(E * t, E * (t + 1))
                vj = vp[rs, cs]
                vs[L * g + RPH * t: L * g + RPH * (t + 1), :] = vj
                tacc[g] = tacc[g] + vj
            elif t == NJ:
                pref[g] = jnp.dot(stril, tacc[g],
                                  preferred_element_type=f32)
            elif t <= 2 * NJ:
                j = t - NJ - 1
                cs = slice(E * j, E * (j + 1))
                run2[g] = run2[g] + vp[rs, cs]
                ctx[L * g + RPH * j: L * g + RPH * (j + 1), :] = (
                    run2[g] + pref[g])

    lane = lax.broadcasted_iota(jnp.int32, (1, L), 1)
    canon_map = ((lane % RPH) * NJ + lane // RPH).astype(f32)
    perm_map = lane.astype(f32)

    for g in range(GH):
        hs = slice(L * g, L * (g + 1))
        rs = slice(RPH * g, RPH * (g + 1))
        selg = sel[U * g: U * (g + 1), :]
        qred = jnp.zeros((U, E), f32)
        for j in range(NJ):
            qred = qred + jnp.dot(selg[:, RPH * j: RPH * (j + 1)],
                                  qp[rs, E * j: E * (j + 1)],
                                  preferred_element_type=f32)
        pos_col = jnp.sum(selg * perm_map, axis=1, keepdims=True)
        pos_i = jnp.clip(pos_col, 0.0, f32(L - 1)).astype(jnp.int32)
        canon_col = ((pos_i % RPH) * NJ + pos_i // RPH).astype(f32)
        scores = lax.dot_general(qred, ks[hs, :], (((1,), (1,)), ((), ())),
                                 preferred_element_type=f32)
        scores = scores * f32(1.0 / math.sqrt(E))
        scores = jnp.where(canon_map > canon_col, neg, scores)
        smax = jnp.max(scores, axis=1, keepdims=True)
        ex = jnp.exp(scores - smax)
        attn = ex / jnp.sum(ex, axis=1, keepdims=True)
        update = lax.dot_general(attn, vs[hs, :], (((1,), (0,)), ((), ())),
                                 preferred_element_type=f32)
        for t in range(U):
            ctx[pl.ds(L * g + pos_i[t, 0], 1), :] = update[t:t + 1, :]

    for g in range(GH):
        flatctx = jnp.concatenate(
            [ctx[L * g + RPH * j: L * g + RPH * (j + 1), :] for j in range(NJ)],
            axis=1)
        out_ref[0, RPH * g: RPH * (g + 1), :] = jnp.dot(
            flatctx, wo_ref[...], preferred_element_type=f32)


@jax.jit
def kernel(q, k, v, Wq, Wk, Wv, Wo):
    _, k2 = jax.random.split(jax.random.key(42))
    kidx = jax.random.randint(k2, (U,), 0, S)
    kidx_p = ((kidx % NJ) * RPH + kidx // NJ).astype(jnp.int32)

    return pl.pallas_call(
        _body,
        grid=(B, H // GH),
        in_specs=[
            pl.BlockSpec(memory_space=pltpu.SMEM),
            pl.BlockSpec((1, ROWS, D_IN), lambda b, hg: (b, hg, 0)),
            pl.BlockSpec((1, ROWS, D_IN), lambda b, hg: (b, hg, 0)),
            pl.BlockSpec((1, ROWS, D_IN), lambda b, hg: (b, hg, 0)),
            pl.BlockSpec((D_IN, HIDDEN), lambda b, hg: (0, 0)),
            pl.BlockSpec((D_IN, HIDDEN), lambda b, hg: (0, 0)),
            pl.BlockSpec((D_IN, HIDDEN), lambda b, hg: (0, 0)),
            pl.BlockSpec((HIDDEN, E), lambda b, hg: (0, 0)),
        ],
        out_specs=pl.BlockSpec((1, ROWS, E), lambda b, hg: (b, hg, 0)),
        out_shape=jax.ShapeDtypeStruct((B, L, E), jnp.float32),
        scratch_shapes=[pltpu.VMEM((GH * L, E), jnp.float32) for _ in range(3)]
        + [pltpu.VMEM((GH * U, L), jnp.float32)],
        compiler_params=pltpu.CompilerParams(
            dimension_semantics=("parallel", "parallel")),
    )(kidx_p, q, k, v, Wq, Wk, Wv, Wo)


# speedup vs baseline: 1.1388x; 1.1388x over previous
import math

import numpy as np
import jax
import jax.numpy as jnp
from jax import lax
from jax.experimental import pallas as pl
from jax.experimental.pallas import tpu as pltpu

B = 4
L = 2048
S = 2048
D_IN = 1024
HIDDEN = 1024
H = 16
E = HIDDEN // H
FACTOR = 5
NJ = HIDDEN // E
RPH = L // NJ
U = min(FACTOR * int(np.ceil(np.log(L))), L)
GH = 4
ROWS = GH * RPH


def _body(kidx_ref, q_ref, k_ref, v_ref, wq_ref, wk_ref, wv_ref, wo_ref,
          out_ref, qs, ks, vs, ctx, sel):
    f32 = jnp.float32
    bf16 = jnp.bfloat16
    qp = jnp.dot(q_ref[0], wq_ref[...], preferred_element_type=f32)
    kp = jnp.dot(k_ref[0], wk_ref[...], preferred_element_type=f32)
    vp = jnp.dot(v_ref[0].astype(bf16), wv_ref[...].astype(bf16),
                 preferred_element_type=f32)

    for g in range(GH):
        rs = slice(RPH * g, RPH * (g + 1))
        for j in range(NJ):
            sl = slice(L * g + RPH * j, L * g + RPH * (j + 1))
            cs = slice(E * j, E * (j + 1))
            qs[sl, :] = qp[rs, cs]
            ks[sl, :] = kp[rs, cs]

    mrows = []
    for g in range(GH):
        hs = slice(L * g, L * (g + 1))
        ksamp = jnp.concatenate(
            [ks[pl.ds(L * g + kidx_ref[t], 1), :] for t in range(U)], axis=0)
        qks = lax.dot_general(ksamp, qs[hs, :], (((1,), (1,)), ((), ())),
                              preferred_element_type=f32)
        mrows.append(jnp.max(qks, axis=0, keepdims=True)
                     - jnp.sum(qks, axis=0, keepdims=True) * f32(1.0 / S))
    cur = jnp.concatenate(mrows, axis=0)

    rio = lax.broadcasted_iota(jnp.int32, (RPH, RPH), 0)
    cio = lax.broadcasted_iota(jnp.int32, (RPH, RPH), 1)
    stril = jnp.where(rio > cio, f32(1.0), f32(0.0))
    neg = f32(-jnp.inf)
    tacc = [jnp.zeros((RPH, E), f32) for _ in range(GH)]
    run2 = [jnp.zeros((RPH, E), f32) for _ in range(GH)]
    pref = [None] * GH
    for t in range(U):
        mval = jnp.max(cur, axis=1, keepdims=True)
        oh = cur == mval
        for g in range(GH):
            sel[U * g + t: U * g + t + 1, :] = jnp.where(
                oh[g:g + 1, :], f32(1.0), f32(0.0))
        cur = jnp.where(oh, neg, cur)
        for g in range(GH):
            rs = slice(RPH * g, RPH * (g + 1))
            if t < NJ:
                cs = slice(E * t, E * (t + 1))
                vj = vp[rs, cs]
                vs[L * g + RPH * t: L * g + RPH * (t + 1), :] = vj
                tacc[g] = tacc[g] + vj
            elif t == NJ:
                pref[g] = jnp.dot(stril, tacc[g],
                                  preferred_element_type=f32)
            elif t <= 2 * NJ:
                j = t - NJ - 1
                cs = slice(E * j, E * (j + 1))
                run2[g] = run2[g] + vp[rs, cs]
                ctx[L * g + RPH * j: L * g + RPH * (j + 1), :] = (
                    run2[g] + pref[g])

    lane = lax.broadcasted_iota(jnp.int32, (1, L), 1)
    canon_map = ((lane % RPH) * NJ + lane // RPH).astype(f32)
    perm_map = lane.astype(f32)

    for g in range(GH):
        hs = slice(L * g, L * (g + 1))
        selg = sel[U * g: U * (g + 1), :]
        qred = jnp.dot(selg, qs[hs, :], preferred_element_type=f32)
        pos_col = jnp.sum(selg * perm_map, axis=1, keepdims=True)
        pos_i = jnp.clip(pos_col, 0.0, f32(L - 1)).astype(jnp.int32)
        canon_col = ((pos_i % RPH) * NJ + pos_i // RPH).astype(f32)
        scores = lax.dot_general(qred, ks[hs, :], (((1,), (1,)), ((), ())),
                                 preferred_element_type=f32)
        scores = scores * f32(1.0 / math.sqrt(E))
        scores = jnp.where(canon_map > canon_col, neg, scores)
        smax = jnp.max(scores, axis=1, keepdims=True)
        ex = jnp.exp(scores - smax)
        attn = ex / jnp.sum(ex, axis=1, keepdims=True)
        update = lax.dot_general(attn, vs[hs, :], (((1,), (0,)), ((), ())),
                                 preferred_element_type=f32)
        for t in range(U):
            ctx[pl.ds(L * g + pos_i[t, 0], 1), :] = update[t:t + 1, :]

    for g in range(GH):
        flatctx = jnp.concatenate(
            [ctx[L * g + RPH * j: L * g + RPH * (j + 1), :] for j in range(NJ)],
            axis=1)
        out_ref[0, RPH * g: RPH * (g + 1), :] = jnp.dot(
            flatctx, wo_ref[...], preferred_element_type=f32)


@jax.jit
def kernel(q, k, v, Wq, Wk, Wv, Wo):
    _, k2 = jax.random.split(jax.random.key(42))
    kidx = jax.random.randint(k2, (U,), 0, S)
    kidx_p = ((kidx % NJ) * RPH + kidx // NJ).astype(jnp.int32)

    return pl.pallas_call(
        _body,
        grid=(B, H // GH),
        in_specs=[
            pl.BlockSpec(memory_space=pltpu.SMEM),
            pl.BlockSpec((1, ROWS, D_IN), lambda b, hg: (b, hg, 0)),
            pl.BlockSpec((1, ROWS, D_IN), lambda b, hg: (b, hg, 0)),
            pl.BlockSpec((1, ROWS, D_IN), lambda b, hg: (b, hg, 0)),
            pl.BlockSpec((D_IN, HIDDEN), lambda b, hg: (0, 0)),
            pl.BlockSpec((D_IN, HIDDEN), lambda b, hg: (0, 0)),
            pl.BlockSpec((D_IN, HIDDEN), lambda b, hg: (0, 0)),
            pl.BlockSpec((HIDDEN, E), lambda b, hg: (0, 0)),
        ],
        out_specs=pl.BlockSpec((1, ROWS, E), lambda b, hg: (b, hg, 0)),
        out_shape=jax.ShapeDtypeStruct((B, L, E), jnp.float32),
        scratch_shapes=[pltpu.VMEM((GH * L, E), jnp.float32) for _ in range(4)]
        + [pltpu.VMEM((GH * U, L), jnp.float32)],
        compiler_params=pltpu.CompilerParams(
            dimension_semantics=("parallel", "parallel")),
    )(kidx_p, q, k, v, Wq, Wk, Wv, Wo)


# speedup vs baseline: 1.2227x; 1.0736x over previous
import math

import numpy as np
import jax
import jax.numpy as jnp
from jax import lax
from jax.experimental import pallas as pl
from jax.experimental.pallas import tpu as pltpu

B = 4
L = 2048
S = 2048
D_IN = 1024
HIDDEN = 1024
H = 16
E = HIDDEN // H
FACTOR = 5
NJ = HIDDEN // E
RPH = L // NJ
U = min(FACTOR * int(np.ceil(np.log(L))), L)
GH = 4
ROWS = GH * RPH


def _body(kidx_ref, q_ref, k_ref, v_ref, wq_ref, wk_ref, wv_ref, wo_ref,
          out_ref, qs, ks, vs, ctx, sel):
    f32 = jnp.float32
    bf16 = jnp.bfloat16
    qp = jnp.dot(q_ref[0], wq_ref[...], preferred_element_type=f32)
    kp = jnp.dot(k_ref[0], wk_ref[...], preferred_element_type=f32)
    vp = jnp.dot(v_ref[0].astype(bf16), wv_ref[...].astype(bf16),
                 preferred_element_type=f32)

    for g in range(GH):
        rs = slice(RPH * g, RPH * (g + 1))
        for j in range(NJ):
            sl = slice(L * g + RPH * j, L * g + RPH * (j + 1))
            cs = slice(E * j, E * (j + 1))
            qs[sl, :] = qp[rs, cs]
            ks[sl, :] = kp[rs, cs]

    mrows = []
    for g in range(GH):
        hs = slice(L * g, L * (g + 1))
        ksamp = jnp.concatenate(
            [ks[pl.ds(L * g + kidx_ref[t], 1), :] for t in range(U)], axis=0)
        qks = lax.dot_general(ksamp, qs[hs, :], (((1,), (1,)), ((), ())),
                              preferred_element_type=f32)
        mrows.append(jnp.max(qks, axis=0, keepdims=True)
                     - jnp.sum(qks, axis=0, keepdims=True) * f32(1.0 / S))
    cur = jnp.concatenate(mrows, axis=0)

    rio = lax.broadcasted_iota(jnp.int32, (RPH, RPH), 0)
    cio = lax.broadcasted_iota(jnp.int32, (RPH, RPH), 1)
    stril = jnp.where(rio > cio, f32(1.0), f32(0.0))
    neg = f32(-jnp.inf)
    tacc = [jnp.zeros((RPH, E), f32) for _ in range(GH)]
    run2 = [jnp.zeros((RPH, E), f32) for _ in range(GH)]
    pref = [None] * GH
    for t in range(U):
        mval = jnp.max(cur, axis=1, keepdims=True)
        oh = cur == mval
        for g in range(GH):
            sel[U * g + t: U * g + t + 1, :] = jnp.where(
                oh[g:g + 1, :], f32(1.0), f32(0.0))
        cur = jnp.where(oh, neg, cur)
        for g in range(GH):
            rs = slice(RPH * g, RPH * (g + 1))
            if t < NJ:
                cs = slice(E * t, E * (t + 1))
                vj = vp[rs, cs]
                vs[L * g + RPH * t: L * g + RPH * (t + 1), :] = vj
                tacc[g] = tacc[g] + vj
            elif t == NJ:
                pref[g] = jnp.dot(stril, tacc[g],
                                  preferred_element_type=f32)
            elif t <= 2 * NJ:
                j = t - NJ - 1
                cs = slice(E * j, E * (j + 1))
                run2[g] = run2[g] + vp[rs, cs]
                ctx[L * g + RPH * j: L * g + RPH * (j + 1), :] = (
                    run2[g] + pref[g])

    lane = lax.broadcasted_iota(jnp.int32, (1, L), 1)
    canon_map = ((lane % RPH) * NJ + lane // RPH).astype(f32)
    perm_map = lane.astype(f32)

    pos_il, masked_l = [], []
    for g in range(GH):
        hs = slice(L * g, L * (g + 1))
        selg = sel[U * g: U * (g + 1), :]
        qred = jnp.dot(selg, qs[hs, :], preferred_element_type=f32)
        pos_col = jnp.sum(selg * perm_map, axis=1, keepdims=True)
        pos_i = jnp.clip(pos_col, 0.0, f32(L - 1)).astype(jnp.int32)
        canon_col = ((pos_i % RPH) * NJ + pos_i // RPH).astype(f32)
        scores = lax.dot_general(qred, ks[hs, :], (((1,), (1,)), ((), ())),
                                 preferred_element_type=f32)
        scores = scores * f32(1.0 / math.sqrt(E))
        pos_il.append(pos_i)
        masked_l.append(jnp.where(canon_map > canon_col, neg, scores))
    attn_l = []
    for g in range(GH):
        scores = masked_l[g]
        smax = jnp.max(scores, axis=1, keepdims=True)
        ex = jnp.exp(scores - smax)
        attn_l.append(ex / jnp.sum(ex, axis=1, keepdims=True))
    for g in range(GH):
        hs = slice(L * g, L * (g + 1))
        update = lax.dot_general(attn_l[g], vs[hs, :], (((1,), (0,)), ((), ())),
                                 preferred_element_type=f32)
        pos_i = pos_il[g]
        for t in range(U):
            ctx[pl.ds(L * g + pos_i[t, 0], 1), :] = update[t:t + 1, :]

    for g in range(GH):
        flatctx = jnp.concatenate(
            [ctx[L * g + RPH * j: L * g + RPH * (j + 1), :] for j in range(NJ)],
            axis=1)
        out_ref[0, RPH * g: RPH * (g + 1), :] = jnp.dot(
            flatctx, wo_ref[...], preferred_element_type=f32)


@jax.jit
def kernel(q, k, v, Wq, Wk, Wv, Wo):
    _, k2 = jax.random.split(jax.random.key(42))
    kidx = jax.random.randint(k2, (U,), 0, S)
    kidx_p = ((kidx % NJ) * RPH + kidx // NJ).astype(jnp.int32)

    return pl.pallas_call(
        _body,
        grid=(B, H // GH),
        in_specs=[
            pl.BlockSpec(memory_space=pltpu.SMEM),
            pl.BlockSpec((1, ROWS, D_IN), lambda b, hg: (b, hg, 0)),
            pl.BlockSpec((1, ROWS, D_IN), lambda b, hg: (b, hg, 0)),
            pl.BlockSpec((1, ROWS, D_IN), lambda b, hg: (b, hg, 0)),
            pl.BlockSpec((D_IN, HIDDEN), lambda b, hg: (0, 0)),
            pl.BlockSpec((D_IN, HIDDEN), lambda b, hg: (0, 0)),
            pl.BlockSpec((D_IN, HIDDEN), lambda b, hg: (0, 0)),
            pl.BlockSpec((HIDDEN, E), lambda b, hg: (0, 0)),
        ],
        out_specs=pl.BlockSpec((1, ROWS, E), lambda b, hg: (b, hg, 0)),
        out_shape=jax.ShapeDtypeStruct((B, L, E), jnp.float32),
        scratch_shapes=[pltpu.VMEM((GH * L, E), jnp.float32) for _ in range(4)]
        + [pltpu.VMEM((GH * U, L), jnp.float32)],
        compiler_params=pltpu.CompilerParams(
            dimension_semantics=("parallel", "parallel")),
    )(kidx_p, q, k, v, Wq, Wk, Wv, Wo)
